# baseline (device time: 47528 ns/iter reference)
import jax
import jax.numpy as jnp
from jax import lax
from jax.experimental import pallas as pl
from jax.experimental.pallas import tpu as pltpu


def kernel(dy, W):
    m, k = dy.shape
    d = W.shape[0]
    half = m // 2

    def body(dy_ref, w_ref, out_ref, ys_ref, yr_ref, xs_ref, xr_ref,
             y_send_sem, y_recv_sem, x_send_sem, x_recv_sem):
        my_x = lax.axis_index("x")
        my_y = lax.axis_index("y")

        barrier_sem = pltpu.get_barrier_semaphore()
        pl.semaphore_signal(barrier_sem, inc=1, device_id=(1 - my_x, my_y),
                            device_id_type=pl.DeviceIdType.MESH)
        pl.semaphore_signal(barrier_sem, inc=1, device_id=(my_x, 1 - my_y),
                            device_id_type=pl.DeviceIdType.MESH)
        pl.semaphore_wait(barrier_sem, 2)

        row0 = my_x * half
        p = lax.dot_general(
            dy_ref[pl.ds(row0, half), :].astype(jnp.bfloat16),
            w_ref[...].astype(jnp.bfloat16),
            dimension_numbers=(((1,), (1,)), ((), ())),
            preferred_element_type=jnp.float32,
        )
        ys_ref[...] = p.astype(jnp.bfloat16)

        rdma_y = pltpu.make_async_remote_copy(
            src_ref=ys_ref, dst_ref=yr_ref,
            send_sem=y_send_sem, recv_sem=y_recv_sem,
            device_id=(my_x, 1 - my_y),
            device_id_type=pl.DeviceIdType.MESH,
        )
        rdma_y.start()
        rdma_y.wait()

        s = ys_ref[...].astype(jnp.float32) + yr_ref[...].astype(jnp.float32)
        out_ref[pl.ds(row0, half), :] = s
        xs_ref[...] = s.astype(jnp.bfloat16)

        rdma_x = pltpu.make_async_remote_copy(
            src_ref=xs_ref, dst_ref=xr_ref,
            send_sem=x_send_sem, recv_sem=x_recv_sem,
            device_id=(1 - my_x, my_y),
            device_id_type=pl.DeviceIdType.MESH,
        )
        rdma_x.start()
        rdma_x.wait()

        out_ref[pl.ds((1 - my_x) * half, half), :] = (
            xr_ref[...].astype(jnp.float32)
        )

    return pl.pallas_call(
        body,
        out_shape=jax.ShapeDtypeStruct((m, d), jnp.float32),
        in_specs=[pl.BlockSpec(memory_space=pltpu.VMEM),
                  pl.BlockSpec(memory_space=pltpu.VMEM)],
        out_specs=pl.BlockSpec(memory_space=pltpu.VMEM),
        scratch_shapes=[
            pltpu.VMEM((half, d), jnp.bfloat16),
            pltpu.VMEM((half, d), jnp.bfloat16),
            pltpu.VMEM((half, d), jnp.bfloat16),
            pltpu.VMEM((half, d), jnp.bfloat16),
            pltpu.SemaphoreType.DMA,
            pltpu.SemaphoreType.DMA,
            pltpu.SemaphoreType.DMA,
            pltpu.SemaphoreType.DMA,
        ],
        compiler_params=pltpu.CompilerParams(collective_id=0),
    )(dy, W)


# device time: 41798 ns/iter; 1.1371x vs baseline; 1.1371x over previous
import jax
import jax.numpy as jnp
from jax import lax
from jax.experimental import pallas as pl
from jax.experimental.pallas import tpu as pltpu

C = 4


def kernel(dy, W):
    m, k = dy.shape
    d = W.shape[0]
    half = m // 2
    rows = half // C

    def body(dy_ref, w_ref, out_ref, wb_ref, ys_ref, yr_ref, xs_ref, xr_ref,
             ys_sems, yr_sems, xs_sems, xr_sems):
        my_x = lax.axis_index("x")
        my_y = lax.axis_index("y")

        def y_rdma(c):
            return pltpu.make_async_remote_copy(
                src_ref=ys_ref.at[c], dst_ref=yr_ref.at[c],
                send_sem=ys_sems.at[c], recv_sem=yr_sems.at[c],
                device_id=(my_x, 1 - my_y),
                device_id_type=pl.DeviceIdType.MESH,
            )

        def x_rdma(c):
            return pltpu.make_async_remote_copy(
                src_ref=xs_ref.at[c], dst_ref=xr_ref.at[c],
                send_sem=xs_sems.at[c], recv_sem=xr_sems.at[c],
                device_id=(1 - my_x, my_y),
                device_id_type=pl.DeviceIdType.MESH,
            )

        barrier_sem = pltpu.get_barrier_semaphore()
        pl.semaphore_signal(barrier_sem, inc=1, device_id=(1 - my_x, my_y),
                            device_id_type=pl.DeviceIdType.MESH)
        pl.semaphore_signal(barrier_sem, inc=1, device_id=(my_x, 1 - my_y),
                            device_id_type=pl.DeviceIdType.MESH)
        pl.semaphore_wait(barrier_sem, 2)

        wb_ref[...] = w_ref[...].astype(jnp.bfloat16)
        row0 = my_x * half

        for c in range(C):
            p = lax.dot_general(
                dy_ref[pl.ds(row0 + c * rows, rows), :].astype(jnp.bfloat16),
                wb_ref[...],
                dimension_numbers=(((1,), (1,)), ((), ())),
                preferred_element_type=jnp.float32,
            )
            ys_ref[c] = p.astype(jnp.bfloat16)
            y_rdma(c).start()

        for c in range(C):
            y_rdma(c).wait_recv()
            s = ys_ref[c].astype(jnp.float32) + yr_ref[c].astype(jnp.float32)
            out_ref[pl.ds(row0 + c * rows, rows), :] = s
            xs_ref[c] = s.astype(jnp.bfloat16)
            x_rdma(c).start()

        other0 = (1 - my_x) * half
        for c in range(C):
            x_rdma(c).wait_recv()
            out_ref[pl.ds(other0 + c * rows, rows), :] = (
                xr_ref[c].astype(jnp.float32)
            )

        for c in range(C):
            y_rdma(c).wait_send()
            x_rdma(c).wait_send()

    return pl.pallas_call(
        body,
        out_shape=jax.ShapeDtypeStruct((m, d), jnp.float32),
        in_specs=[pl.BlockSpec(memory_space=pltpu.VMEM),
                  pl.BlockSpec(memory_space=pltpu.VMEM)],
        out_specs=pl.BlockSpec(memory_space=pltpu.VMEM),
        scratch_shapes=[
            pltpu.VMEM((d, k), jnp.bfloat16),
            pltpu.VMEM((C, rows, d), jnp.bfloat16),
            pltpu.VMEM((C, rows, d), jnp.bfloat16),
            pltpu.VMEM((C, rows, d), jnp.bfloat16),
            pltpu.VMEM((C, rows, d), jnp.bfloat16),
            pltpu.SemaphoreType.DMA((C,)),
            pltpu.SemaphoreType.DMA((C,)),
            pltpu.SemaphoreType.DMA((C,)),
            pltpu.SemaphoreType.DMA((C,)),
        ],
        compiler_params=pltpu.CompilerParams(collective_id=0),
    )(dy, W)
